# baseline (speedup 1.0000x reference)
import functools
import math

import jax
import jax.numpy as jnp
from jax import lax
from jax.experimental import pallas as pl
from jax.experimental.pallas import tpu as pltpu
from jax.experimental.pallas import tpu_sc as plsc

ENTITY_NUM = 100000
RELATION_NUM = 1000
DIM = 64
BATCH = 16384
PATHS_PER_PAIR = 4
PATH_LEN = 3
GAMMA = 1.0

NC = 2
NS = 16
L = 16
NW = NC * NS
W = BATCH // NW
C = 64
NCHUNK = W // C
CP = C * PATHS_PER_PAIR


def _fast_sqrt(s):
    x = jnp.maximum(s, 1e-30)
    i = plsc.bitcast(x, jnp.int32)
    i = jnp.full((L,), 0x5F3759DF, jnp.int32) - lax.shift_right_logical(i, 1)
    y = plsc.bitcast(i, jnp.float32)
    half = 0.5 * x
    for _ in range(3):
        y = y * (1.5 - half * y * y)
    return x * y


def _body(ent_hbm, rel_hbm, typ_hbm, probs_hbm,
          ph_hbm, pr_hbm, pt_hbm, nh_hbm, nr_hbm, nt_hbm, tok_hbm,
          out_hbm,
          ehb, thb, etb, ttb, nehb, nthb, netb, nttb, rpb, nrb, ppb,
          phv, ptv, nhv, ntv, prv, nrv, tokv, probv, accv, sem, semt):
    cid = lax.axis_index("c")
    sid = lax.axis_index("s")
    wid = sid * NC + cid

    accv[...] = jnp.zeros((L,), jnp.float32)

    for t in range(PATH_LEN):
        pltpu.sync_copy(tok_hbm.at[t, pl.ds(wid * (W * 4 // 128), W * 4 // 128)],
                        tokv.at[pl.ds(t * (W * 4 // 128), W * 4 // 128)])

    def chunk_body(ch, _):
        base = wid * W + ch * C

        pltpu.sync_copy(ph_hbm.at[pl.ds(base, C)], phv)
        pltpu.sync_copy(pt_hbm.at[pl.ds(base, C)], ptv)
        pltpu.sync_copy(nh_hbm.at[pl.ds(base, C)], nhv)
        pltpu.sync_copy(nt_hbm.at[pl.ds(base, C)], ntv)
        pltpu.sync_copy(pr_hbm.at[pl.ds(base, C)], prv)
        pltpu.sync_copy(nr_hbm.at[pl.ds(base, C)], nrv)
        rows_per_w = W * 4 // 128
        tok0 = [pltpu.async_copy(rel_hbm.at[tokv.at[ch * 2 + j]],
                                 ppb.at[pl.ds(j * 128, 128)], semt)
                for j in range(2)]
        descs = [
            pltpu.async_copy(probs_hbm.at[pl.ds(base * 4, CP)], probv, sem),
            pltpu.async_copy(ent_hbm.at[phv], ehb, sem),
            pltpu.async_copy(typ_hbm.at[phv], thb, sem),
            pltpu.async_copy(ent_hbm.at[ptv], etb, sem),
            pltpu.async_copy(typ_hbm.at[ptv], ttb, sem),
            pltpu.async_copy(ent_hbm.at[nhv], nehb, sem),
            pltpu.async_copy(typ_hbm.at[nhv], nthb, sem),
            pltpu.async_copy(ent_hbm.at[ntv], netb, sem),
            pltpu.async_copy(typ_hbm.at[ntv], nttb, sem),
            pltpu.async_copy(rel_hbm.at[prv], rpb, sem),
            pltpu.async_copy(rel_hbm.at[nrv], nrb, sem),
        ]
        for d in tok0:
            d.wait()
        adds = [
            pltpu.async_copy(rel_hbm.at[tokv.at[t * rows_per_w + ch * 2 + j]],
                             ppb.at[pl.ds(j * 128, 128)], semt, add=True)
            for t in range(1, PATH_LEN) for j in range(2)
        ]
        for d in descs:
            d.wait()
        for d in adds:
            d.wait()

        def group_body(g, _):
            lane = lax.iota(jnp.int32, 16)
            rl = lane + g * L
            rl4 = rl * 4
            pr0 = plsc.load_gather(probv, [rl4])
            pr1 = plsc.load_gather(probv, [rl4 + 1])
            pr2 = plsc.load_gather(probv, [rl4 + 2])
            pr3 = plsc.load_gather(probv, [rl4 + 3])

            z = jnp.zeros((L,), jnp.float32)

            @plsc.parallel_loop(0, DIM, 1, unroll=4, carry=(z, z))
            def c_loop(c, carry):
                s_pos, s_neg = carry
                cv = jnp.bitwise_and(c + lane, DIM - 1)
                eh = plsc.load_gather(ehb, [rl, cv])
                th = plsc.load_gather(thb, [rl, cv])
                et = plsc.load_gather(etb, [rl, cv])
                tt = plsc.load_gather(ttb, [rl, cv])
                neh = plsc.load_gather(nehb, [rl, cv])
                nth = plsc.load_gather(nthb, [rl, cv])
                net = plsc.load_gather(netb, [rl, cv])
                ntt = plsc.load_gather(nttb, [rl, cv])
                rp = plsc.load_gather(rpb, [rl, cv])
                nr = plsc.load_gather(nrb, [rl, cv])
                p0 = plsc.load_gather(ppb, [rl4, cv])
                p1 = plsc.load_gather(ppb, [rl4 + 1, cv])
                p2 = plsc.load_gather(ppb, [rl4 + 2, cv])
                p3 = plsc.load_gather(ppb, [rl4 + 3, cv])
                pf = pr0 * p0 + pr1 * p1 + pr2 * p2 + pr3 * p3
                pos = eh * th + rp + pf - et * tt
                neg = neh * nth + nr - net * ntt
                return s_pos + pos * pos, s_neg + neg * neg

            s_pos, s_neg = c_loop
            pn = _fast_sqrt(s_pos)
            nn = _fast_sqrt(s_neg)
            dd = GAMMA + pn - nn
            contrib = jnp.maximum(dd, 0.0) + 0.001 * (pn + nn)
            accv[...] = accv[...] + contrib
            return 0

        lax.fori_loop(0, C // L, group_body, 0)
        return 0

    lax.fori_loop(0, NCHUNK, chunk_body, 0)
    pltpu.sync_copy(accv, out_hbm.at[pl.ds(wid * L, L)])


@jax.jit
def _run(entity_emb, relation_emb, type_emb, path_probs,
         pos_head, pos_relation, pos_tail,
         neg_head, neg_relation, neg_tail, tok_t):
    mesh = plsc.VectorSubcoreMesh(core_axis_name="c", subcore_axis_name="s",
                                  num_cores=NC, num_subcores=NS)
    kern = pl.kernel(
        _body,
        out_type=jax.ShapeDtypeStruct((NW * L,), jnp.float32),
        mesh=mesh,
        compiler_params=pltpu.CompilerParams(
            needs_layout_passes=False, use_tc_tiling_on_sc=False),
        scratch_types=[
            pltpu.VMEM((C, DIM), jnp.float32),
            pltpu.VMEM((C, DIM), jnp.float32),
            pltpu.VMEM((C, DIM), jnp.float32),
            pltpu.VMEM((C, DIM), jnp.float32),
            pltpu.VMEM((C, DIM), jnp.float32),
            pltpu.VMEM((C, DIM), jnp.float32),
            pltpu.VMEM((C, DIM), jnp.float32),
            pltpu.VMEM((C, DIM), jnp.float32),
            pltpu.VMEM((C, DIM), jnp.float32),
            pltpu.VMEM((C, DIM), jnp.float32),
            pltpu.VMEM((CP, DIM), jnp.float32),
            pltpu.VMEM((C,), jnp.int32),
            pltpu.VMEM((C,), jnp.int32),
            pltpu.VMEM((C,), jnp.int32),
            pltpu.VMEM((C,), jnp.int32),
            pltpu.VMEM((C,), jnp.int32),
            pltpu.VMEM((C,), jnp.int32),
            pltpu.VMEM((PATH_LEN * (W * 4 // 128), 128), jnp.int32),
            pltpu.VMEM((CP,), jnp.float32),
            pltpu.VMEM((L,), jnp.float32),
            pltpu.SemaphoreType.DMA,
            pltpu.SemaphoreType.DMA,
        ],
    )
    partials = kern(entity_emb, relation_emb, type_emb, path_probs,
                    pos_head, pos_relation, pos_tail,
                    neg_head, neg_relation, neg_tail, tok_t)
    return jnp.sum(partials)


def kernel(entity_emb, relation_emb, type_emb, path_probs,
           pos_head, pos_relation, pos_tail,
           neg_head, neg_relation, neg_tail, path_rel_idx):
    tok_t = (path_rel_idx.astype(jnp.int32)
             .reshape(BATCH * PATHS_PER_PAIR, PATH_LEN)
             .T.reshape(PATH_LEN, BATCH * PATHS_PER_PAIR // 128, 128))
    return _run(entity_emb, relation_emb, type_emb, path_probs,
                pos_head.astype(jnp.int32), pos_relation.astype(jnp.int32),
                pos_tail.astype(jnp.int32), neg_head.astype(jnp.int32),
                neg_relation.astype(jnp.int32), neg_tail.astype(jnp.int32),
                tok_t)


# speedup vs baseline: 1.1459x; 1.1459x over previous
import functools
import math

import jax
import jax.numpy as jnp
from jax import lax
from jax.experimental import pallas as pl
from jax.experimental.pallas import tpu as pltpu
from jax.experimental.pallas import tpu_sc as plsc

ENTITY_NUM = 100000
RELATION_NUM = 1000
DIM = 64
BATCH = 16384
PATHS_PER_PAIR = 4
PATH_LEN = 3
GAMMA = 1.0

NC = 2
NS = 16
L = 16
NW = NC * NS
W = BATCH // NW
C = 64
NCHUNK = W // C
CP = C * PATHS_PER_PAIR
CT = C * PATHS_PER_PAIR * PATH_LEN


def _fast_sqrt(s):
    x = jnp.maximum(s, 1e-30)
    i = plsc.bitcast(x, jnp.int32)
    i = jnp.full((L,), 0x5F3759DF, jnp.int32) - lax.shift_right_logical(i, 1)
    y = plsc.bitcast(i, jnp.float32)
    half = 0.5 * x
    for _ in range(3):
        y = y * (1.5 - half * y * y)
    return x * y


def _body(ent_hbm, rel_hbm, typ_hbm, probs_hbm,
          ph_hbm, pr_hbm, pt_hbm, nh_hbm, nr_hbm, nt_hbm, tok_hbm,
          out_hbm,
          relv, ehb, thb, etb, ttb, nehb, nthb, netb, nttb,
          phv, ptv, nhv, ntv, prv, nrv, tokv, probv, accv, sem):
    cid = lax.axis_index("c")
    sid = lax.axis_index("s")
    wid = sid * NC + cid

    accv[...] = jnp.zeros((L,), jnp.float32)
    pltpu.sync_copy(rel_hbm, relv)

    def chunk_body(ch, _):
        base = wid * W + ch * C

        pltpu.sync_copy(ph_hbm.at[pl.ds(base, C)], phv)
        pltpu.sync_copy(pt_hbm.at[pl.ds(base, C)], ptv)
        pltpu.sync_copy(nh_hbm.at[pl.ds(base, C)], nhv)
        pltpu.sync_copy(nt_hbm.at[pl.ds(base, C)], ntv)

        descs = [
            pltpu.async_copy(pr_hbm.at[pl.ds(base, C)], prv, sem),
            pltpu.async_copy(nr_hbm.at[pl.ds(base, C)], nrv, sem),
            pltpu.async_copy(tok_hbm.at[pl.ds(base * 12, CT)], tokv, sem),
            pltpu.async_copy(probs_hbm.at[pl.ds(base * 4, CP)], probv, sem),
            pltpu.async_copy(ent_hbm.at[phv], ehb, sem),
            pltpu.async_copy(typ_hbm.at[phv], thb, sem),
            pltpu.async_copy(ent_hbm.at[ptv], etb, sem),
            pltpu.async_copy(typ_hbm.at[ptv], ttb, sem),
            pltpu.async_copy(ent_hbm.at[nhv], nehb, sem),
            pltpu.async_copy(typ_hbm.at[nhv], nthb, sem),
            pltpu.async_copy(ent_hbm.at[ntv], netb, sem),
            pltpu.async_copy(typ_hbm.at[ntv], nttb, sem),
        ]
        for d in descs:
            d.wait()

        def group_body(g, loss16):
            lane = lax.iota(jnp.int32, 16)
            rl = lane + g * L
            rl4 = rl * 4
            rl12 = rl * 12
            pr0 = plsc.load_gather(probv, [rl4])
            pr1 = plsc.load_gather(probv, [rl4 + 1])
            pr2 = plsc.load_gather(probv, [rl4 + 2])
            pr3 = plsc.load_gather(probv, [rl4 + 3])
            pridx = plsc.load_gather(prv, [rl])
            nridx = plsc.load_gather(nrv, [rl])
            trow = [plsc.load_gather(tokv, [rl12 + k]) for k in range(12)]

            z = jnp.zeros((L,), jnp.float32)

            @plsc.parallel_loop(0, DIM, 1, unroll=4, carry=(z, z))
            def c_loop(c, carry):
                s_pos, s_neg = carry
                cv = jnp.bitwise_and(c + lane, DIM - 1)
                eh = plsc.load_gather(ehb, [rl, cv])
                th = plsc.load_gather(thb, [rl, cv])
                et = plsc.load_gather(etb, [rl, cv])
                tt = plsc.load_gather(ttb, [rl, cv])
                neh = plsc.load_gather(nehb, [rl, cv])
                nth = plsc.load_gather(nthb, [rl, cv])
                net = plsc.load_gather(netb, [rl, cv])
                ntt = plsc.load_gather(nttb, [rl, cv])
                rp = plsc.load_gather(relv, [pridx, cv])
                nr = plsc.load_gather(relv, [nridx, cv])
                t = [plsc.load_gather(relv, [trow[k], cv]) for k in range(12)]
                s0 = t[0] + t[1] + t[2]
                s1 = t[3] + t[4] + t[5]
                s2 = t[6] + t[7] + t[8]
                s3 = t[9] + t[10] + t[11]
                pf = pr0 * s0 + pr1 * s1 + pr2 * s2 + pr3 * s3
                pos = eh * th + rp + pf - et * tt
                neg = neh * nth + nr - net * ntt
                return s_pos + pos * pos, s_neg + neg * neg

            s_pos, s_neg = c_loop
            pn = _fast_sqrt(s_pos)
            nn = _fast_sqrt(s_neg)
            dd = GAMMA + pn - nn
            return loss16 + jnp.maximum(dd, 0.0) + 0.001 * (pn + nn)

        loss16 = lax.fori_loop(0, C // L, group_body,
                               jnp.zeros((L,), jnp.float32))
        accv[...] = accv[...] + loss16
        return 0

    lax.fori_loop(0, NCHUNK, chunk_body, 0)
    pltpu.sync_copy(accv, out_hbm.at[pl.ds(wid * L, L)])


@jax.jit
def _run(entity_emb, relation_emb, type_emb, path_probs,
         pos_head, pos_relation, pos_tail,
         neg_head, neg_relation, neg_tail, path_rel_idx):
    mesh = plsc.VectorSubcoreMesh(core_axis_name="c", subcore_axis_name="s",
                                  num_cores=NC, num_subcores=NS)
    kern = pl.kernel(
        _body,
        out_type=jax.ShapeDtypeStruct((NW * L,), jnp.float32),
        mesh=mesh,
        compiler_params=pltpu.CompilerParams(
            needs_layout_passes=False, use_tc_tiling_on_sc=False),
        scratch_types=[
            pltpu.VMEM((RELATION_NUM, DIM), jnp.float32),
            pltpu.VMEM((C, DIM), jnp.float32),
            pltpu.VMEM((C, DIM), jnp.float32),
            pltpu.VMEM((C, DIM), jnp.float32),
            pltpu.VMEM((C, DIM), jnp.float32),
            pltpu.VMEM((C, DIM), jnp.float32),
            pltpu.VMEM((C, DIM), jnp.float32),
            pltpu.VMEM((C, DIM), jnp.float32),
            pltpu.VMEM((C, DIM), jnp.float32),
            pltpu.VMEM((C,), jnp.int32),
            pltpu.VMEM((C,), jnp.int32),
            pltpu.VMEM((C,), jnp.int32),
            pltpu.VMEM((C,), jnp.int32),
            pltpu.VMEM((C,), jnp.int32),
            pltpu.VMEM((C,), jnp.int32),
            pltpu.VMEM((CT,), jnp.int32),
            pltpu.VMEM((CP,), jnp.float32),
            pltpu.VMEM((L,), jnp.float32),
            pltpu.SemaphoreType.DMA,
        ],
    )
    partials = kern(entity_emb, relation_emb, type_emb, path_probs,
                    pos_head, pos_relation, pos_tail,
                    neg_head, neg_relation, neg_tail, path_rel_idx)
    return jnp.sum(partials)


def kernel(entity_emb, relation_emb, type_emb, path_probs,
           pos_head, pos_relation, pos_tail,
           neg_head, neg_relation, neg_tail, path_rel_idx):
    return _run(entity_emb, relation_emb, type_emb, path_probs,
                pos_head.astype(jnp.int32), pos_relation.astype(jnp.int32),
                pos_tail.astype(jnp.int32), neg_head.astype(jnp.int32),
                neg_relation.astype(jnp.int32), neg_tail.astype(jnp.int32),
                path_rel_idx.astype(jnp.int32))
